# speedup vs baseline: 1.2672x; 1.2672x over previous
import functools
import jax
import jax.numpy as jnp
from jax import lax
from jax.experimental import pallas as pl
from jax.experimental.pallas import tpu as pltpu

N = 10000
E = 320000
DIN = 128
DH = 128
DOUT = 64
K = 6
EPS = 1e-07
BN_EPS = 1e-05

BN_ROWS = 1000


def _combine_body(x_ref, hlo_ref, hhi_ref, w0_ref, wlo_ref, whi_ref,
                  m1w_ref, m1b_ref, m2w_ref, m2b_ref, out_ref, *, relu_out):
    acc = jnp.dot(x_ref[...], w0_ref[...], preferred_element_type=jnp.float32)
    for k in range(K):
        acc += jnp.dot(hlo_ref[k], wlo_ref[k], preferred_element_type=jnp.float32)
        acc += jnp.dot(hhi_ref[k], whi_ref[k], preferred_element_type=jnp.float32)
    y = jnp.dot(acc, m1w_ref[...], preferred_element_type=jnp.float32) + m1b_ref[...]
    y = jnp.maximum(y, 0.0)
    o = jnp.dot(y, m2w_ref[...], preferred_element_type=jnp.float32) + m2b_ref[...]
    if relu_out:
        o = jnp.maximum(o, 0.0)
    out_ref[...] = o


def _combine(x, hlo, hhi, lins, bias, m1w, m1b, bng, bnb, m2w, m2b, relu_out):
    d = lins.shape[1]
    dh = lins.shape[2]
    d2 = m1w.shape[1]
    do = m2w.shape[1]
    rs = jax.lax.rsqrt(jnp.asarray(1.0 + BN_EPS, jnp.float32)) * bng
    m1w_f = m1w * rs[None, :]
    m1b_f = (bias @ m1w) * rs + m1b * rs + bnb
    w0 = lins[0]
    wlo = lins[1:, :64, :]
    whi = lins[1:, 64:, :]

    grid = (N // BN_ROWS,)
    return pl.pallas_call(
        functools.partial(_combine_body, relu_out=relu_out),
        grid=grid,
        in_specs=[
            pl.BlockSpec((BN_ROWS, d), lambda i: (i, 0)),
            pl.BlockSpec((K, BN_ROWS, 64), lambda i: (0, i, 0)),
            pl.BlockSpec((K, BN_ROWS, 64), lambda i: (0, i, 0)),
            pl.BlockSpec((d, dh), lambda i: (0, 0)),
            pl.BlockSpec((K, 64, dh), lambda i: (0, 0, 0)),
            pl.BlockSpec((K, 64, dh), lambda i: (0, 0, 0)),
            pl.BlockSpec((dh, d2), lambda i: (0, 0)),
            pl.BlockSpec((1, d2), lambda i: (0, 0)),
            pl.BlockSpec((d2, do), lambda i: (0, 0)),
            pl.BlockSpec((1, do), lambda i: (0, 0)),
        ],
        out_specs=pl.BlockSpec((BN_ROWS, do), lambda i: (i, 0)),
        out_shape=jax.ShapeDtypeStruct((N, do), jnp.float32),
    )(x, hlo, hhi, w0, wlo, whi, m1w_f, m1b_f.reshape(1, d2), m2w,
      m2b.reshape(1, do))


def _hops_xla(x, row, col, deg, dinv):
    hlo = []
    hhi = []
    h = x
    epsdeg = EPS * deg[:, None]
    for _ in range(K):
        g = dinv[:, None] * jnp.maximum(h, 0.0)
        h = dinv[:, None] * jax.ops.segment_sum(
            jnp.take(g, row, axis=0), col, num_segments=N) + epsdeg
        hlo.append(h[:, :64])
        hhi.append(h[:, 64:])
    return jnp.stack(hlo), jnp.stack(hhi)


def kernel(x, edge_index, c1_lins, c1_bias, c1_m1w, c1_m1b, c1_bng, c1_bnb,
           c1_m2w, c1_m2b, c2_lins, c2_bias, c2_m1w, c2_m1b, c2_bng, c2_bnb,
           c2_m2w, c2_m2b):
    row = edge_index[0]
    col = edge_index[1]
    deg = jax.ops.segment_sum(jnp.ones((E,), jnp.float32), col, num_segments=N)
    dinv = jnp.where(deg > 0, jax.lax.rsqrt(jnp.maximum(deg, 1e-12)), 0.0)

    hlo, hhi = _hops_xla(x, row, col, deg, dinv)
    h = _combine(x, hlo, hhi, c1_lins, c1_bias, c1_m1w, c1_m1b, c1_bng, c1_bnb,
                 c1_m2w, c1_m2b, relu_out=True)
    hlo, hhi = _hops_xla(h, row, col, deg, dinv)
    out = _combine(h, hlo, hhi, c2_lins, c2_bias, c2_m1w, c2_m1b, c2_bng,
                   c2_bnb, c2_m2w, c2_m2b, relu_out=False)
    return out


# speedup vs baseline: 3.2759x; 2.5851x over previous
import functools
import jax
import jax.numpy as jnp
from jax import lax
from jax.experimental import pallas as pl
from jax.experimental.pallas import tpu as pltpu
from jax.experimental.pallas import tpu_sc as plsc

N = 10000
E = 320000
DIN = 128
DH = 128
DOUT = 64
K = 6
EPS = 1e-07
BN_EPS = 1e-05

NC = 2
NS = 16
L = 16
W = 128

NP = 10240
RT = NP // NS
FB = 64
C = 80
EPT = E // NS
NCH = EPT // C

BN_ROWS = 1000


def _sc_deg(col_e):
    mesh = plsc.VectorSubcoreMesh(core_axis_name="c", subcore_axis_name="s")
    ept2 = E // (NC * NS)
    nch2 = ept2 // C

    @functools.partial(
        pl.kernel,
        out_type=jax.ShapeDtypeStruct((2 * NP,), jnp.float32),
        mesh=mesh,
        scratch_types=[
            pltpu.VMEM_SHARED((NP,), jnp.float32),
            pltpu.VMEM((1, C), jnp.int32),
            pltpu.VMEM((C,), jnp.float32),
            pltpu.VMEM((RT,), jnp.float32),
        ],
    )
    def degk(col_hbm, deg_hbm, deg_sh, colidx, ones_buf, stage):
        c = lax.axis_index("c")
        s = lax.axis_index("s")
        node0 = s * RT
        e_base = (c * NS + s) * ept2

        @pl.loop(0, C // L)
        def _(j):
            ones_buf[pl.ds(j * L, L)] = jnp.ones((L,), jnp.float32)

        @pl.loop(0, RT // L)
        def _(j):
            stage[pl.ds(j * L, L)] = jnp.zeros((L,), jnp.float32)

        pltpu.sync_copy(stage, deg_sh.at[pl.ds(node0, RT)])
        plsc.subcore_barrier()

        @pl.loop(0, nch2)
        def _(ch):
            e0 = e_base + ch * C
            pltpu.sync_copy(col_hbm.at[pl.ds(e0, C)], colidx.at[0])
            pltpu.sync_copy(ones_buf, deg_sh.at[colidx.at[0]], add=True)
        plsc.subcore_barrier()

        pltpu.sync_copy(deg_sh.at[pl.ds(node0, RT)], stage)
        pltpu.sync_copy(stage, deg_hbm.at[pl.ds(c * NP + node0, RT)])

    return degk(col_e)


def _tc_prep_body(deg2_ref, out_ref):
    d = deg2_ref[0:1, :] + deg2_ref[1:2, :]
    dv = jnp.where(d > 0.0, jax.lax.rsqrt(jnp.maximum(d, 1e-12)), 0.0)
    out_ref[0:1, :] = dv
    out_ref[1:2, :] = d * EPS


def _tc_prep(deg2):
    out = pl.pallas_call(
        _tc_prep_body,
        out_shape=jax.ShapeDtypeStruct((2, NP), jnp.float32),
    )(deg2.reshape(2, NP))
    return out.reshape(2 * NP)


def _sc_hops(src_pad, row_e, col_e, dep):
    mesh = plsc.VectorSubcoreMesh(core_axis_name="c", subcore_axis_name="s",
                                  num_cores=1)

    @functools.partial(
        pl.kernel,
        out_type=[
            jax.ShapeDtypeStruct((K * NP, W), jnp.float32),
            jax.ShapeDtypeStruct((NP, W), jnp.float32),
        ],
        mesh=mesh,
        scratch_types=[
            pltpu.VMEM_SHARED((NP, W), jnp.float32),
            pltpu.VMEM((1, C), jnp.int32),
            pltpu.VMEM((1, C), jnp.int32),
            pltpu.VMEM((C, W), jnp.float32),
            pltpu.VMEM((FB, W), jnp.float32),
            pltpu.VMEM((FB, W), jnp.float32),
            pltpu.VMEM((FB, W), jnp.float32),
            pltpu.VMEM((RT,), jnp.float32),
            pltpu.VMEM((RT,), jnp.float32),
        ],
        compiler_params=pltpu.CompilerParams(use_tc_tiling_on_sc=False),
    )
    def hops(src_hbm, row_hbm, col_hbm, dep_hbm, h_hbm, g_hbm, acc_sh,
             rowidx, colidx, rows_buf, flush_buf, g_buf, zero_buf,
             dinv_t, epsdeg_t):
        s = lax.axis_index("s")
        node0 = s * RT
        e_base = s * EPT

        @pl.loop(0, FB)
        def _(i):
            for j in range(W // L):
                zero_buf[i, pl.ds(j * L, L)] = jnp.zeros((L,), jnp.float32)

        @pl.loop(0, RT // FB)
        def _(sub):
            pltpu.sync_copy(zero_buf, acc_sh.at[pl.ds(node0 + sub * FB, FB)])

        pltpu.sync_copy(dep_hbm.at[pl.ds(node0, RT)], dinv_t)
        pltpu.sync_copy(dep_hbm.at[pl.ds(NP + node0, RT)], epsdeg_t)

        @pl.loop(0, RT // FB)
        def _(sub):
            r0 = node0 + sub * FB
            pltpu.sync_copy(src_hbm.at[pl.ds(r0, FB)], flush_buf)

            @pl.loop(0, FB // L)
            def _(g):
                dv16 = dinv_t[pl.ds(sub * FB + g * L, L)]
                for i2 in range(L):
                    i = g * L + i2
                    dv = dv16[i2]
                    for j in range(W // L):
                        v = flush_buf[i, pl.ds(j * L, L)]
                        g_buf[i, pl.ds(j * L, L)] = jnp.maximum(v, 0.0) * dv

            pltpu.sync_copy(g_buf, g_hbm.at[pl.ds(r0, FB)])

        @pl.loop(1, K + 1)
        def _(k):
            plsc.subcore_barrier()

            @pl.loop(0, NCH)
            def _(ch):
                e0 = e_base + ch * C
                pltpu.sync_copy(row_hbm.at[pl.ds(e0, C)], rowidx.at[0])
                pltpu.sync_copy(col_hbm.at[pl.ds(e0, C)], colidx.at[0])
                pltpu.sync_copy(g_hbm.at[rowidx.at[0]], rows_buf)
                pltpu.sync_copy(rows_buf, acc_sh.at[colidx.at[0]], add=True)

            plsc.subcore_barrier()

            @pl.loop(0, RT // FB)
            def _(sub):
                r0 = node0 + sub * FB
                pltpu.sync_copy(acc_sh.at[pl.ds(r0, FB)], flush_buf)
                pltpu.sync_copy(zero_buf, acc_sh.at[pl.ds(r0, FB)])

                @pl.loop(0, FB // L)
                def _(g):
                    dv16 = dinv_t[pl.ds(sub * FB + g * L, L)]
                    ed16 = epsdeg_t[pl.ds(sub * FB + g * L, L)]
                    for i2 in range(L):
                        i = g * L + i2
                        dv = dv16[i2]
                        ed = ed16[i2]
                        for j in range(W // L):
                            h16 = flush_buf[i, pl.ds(j * L, L)] * dv + ed
                            flush_buf[i, pl.ds(j * L, L)] = h16
                            g_buf[i, pl.ds(j * L, L)] = jnp.maximum(h16, 0.0) * dv

                hoff = (k - 1) * NP + r0
                pltpu.sync_copy(flush_buf, h_hbm.at[pl.ds(hoff, FB)])
                pltpu.sync_copy(g_buf, g_hbm.at[pl.ds(r0, FB)])

    return hops(src_pad, row_e, col_e, dep)


def _pad_rows(h):
    return jnp.zeros((NP, W), h.dtype).at[:N].set(h)


def _combine_body(x_ref, hs_ref, w0_ref, ws_ref,
                  m1w_ref, m1b_ref, m2w_ref, m2b_ref, out_ref, *, relu_out):
    acc = jnp.dot(x_ref[...], w0_ref[...], preferred_element_type=jnp.float32)
    for k in range(K):
        acc += jnp.dot(hs_ref[k], ws_ref[k], preferred_element_type=jnp.float32)
    y = jnp.dot(acc, m1w_ref[...], preferred_element_type=jnp.float32) + m1b_ref[...]
    y = jnp.maximum(y, 0.0)
    o = jnp.dot(y, m2w_ref[...], preferred_element_type=jnp.float32) + m2b_ref[...]
    if relu_out:
        o = jnp.maximum(o, 0.0)
    out_ref[...] = o


def _combine(x, hs, lins, bias, m1w, m1b, bng, bnb, m2w, m2b, relu_out):
    d = lins.shape[1]
    dh = lins.shape[2]
    d2 = m1w.shape[1]
    do = m2w.shape[1]
    rs = jax.lax.rsqrt(jnp.asarray(1.0 + BN_EPS, jnp.float32)) * bng
    m1w_f = m1w * rs[None, :]
    m1b_f = (bias @ m1w) * rs + m1b * rs + bnb
    w0 = lins[0]
    ws = lins[1:]

    grid = (N // BN_ROWS,)
    return pl.pallas_call(
        functools.partial(_combine_body, relu_out=relu_out),
        grid=grid,
        in_specs=[
            pl.BlockSpec((BN_ROWS, d), lambda i: (i, 0)),
            pl.BlockSpec((K, BN_ROWS, d), lambda i: (0, i, 0)),
            pl.BlockSpec((d, dh), lambda i: (0, 0)),
            pl.BlockSpec((K, d, dh), lambda i: (0, 0, 0)),
            pl.BlockSpec((dh, d2), lambda i: (0, 0)),
            pl.BlockSpec((1, d2), lambda i: (0, 0)),
            pl.BlockSpec((d2, do), lambda i: (0, 0)),
            pl.BlockSpec((1, do), lambda i: (0, 0)),
        ],
        out_specs=pl.BlockSpec((BN_ROWS, do), lambda i: (i, 0)),
        out_shape=jax.ShapeDtypeStruct((N, do), jnp.float32),
    )(x, hs, w0, ws, m1w_f, m1b_f.reshape(1, d2), m2w, m2b.reshape(1, do))


def _layer(x, row_e, col_e, dep, lins, bias, m1w, m1b, bng, bnb, m2w, m2b,
           relu_out):
    h_flat, _ = _sc_hops(_pad_rows(x), row_e, col_e, dep)
    hs = h_flat.reshape(K, NP, W)[:, :N, :]
    return _combine(x, hs, lins, bias, m1w, m1b, bng, bnb, m2w, m2b, relu_out)


def kernel(x, edge_index, c1_lins, c1_bias, c1_m1w, c1_m1b, c1_bng, c1_bnb,
           c1_m2w, c1_m2b, c2_lins, c2_bias, c2_m1w, c2_m1b, c2_bng, c2_bnb,
           c2_m2w, c2_m2b):
    row_e = edge_index[0]
    col_e = edge_index[1]
    dep = _tc_prep(_sc_deg(col_e))
    h = _layer(x, row_e, col_e, dep, c1_lins, c1_bias, c1_m1w, c1_m1b, c1_bng,
               c1_bnb, c1_m2w, c1_m2b, relu_out=True)
    return _layer(h, row_e, col_e, dep, c2_lins, c2_bias, c2_m1w, c2_m1b,
                  c2_bng, c2_bnb, c2_m2w, c2_m2b, relu_out=False)


# speedup vs baseline: 7.0233x; 2.1439x over previous
import functools
import jax
import jax.numpy as jnp
from jax import lax
from jax.experimental import pallas as pl
from jax.experimental.pallas import tpu as pltpu
from jax.experimental.pallas import tpu_sc as plsc

N = 10000
E = 320000
DIN = 128
DH = 128
DOUT = 64
K = 6
EPS = 1e-07
BN_EPS = 1e-05

NC = 2
NS = 16
L = 16
W = 128

NP = 10240
RT = NP // NS
FB = 32
C = 100
EPT = E // NS
NCH = EPT // C
GB = 10
NBLK = NCH // GB

BN_ROWS = 1000


def _sc_deg(col_e):
    mesh = plsc.VectorSubcoreMesh(core_axis_name="c", subcore_axis_name="s")
    cd = 80
    ept2 = E // (NC * NS)
    nch2 = ept2 // cd

    @functools.partial(
        pl.kernel,
        out_type=jax.ShapeDtypeStruct((2 * NP,), jnp.float32),
        mesh=mesh,
        scratch_types=[
            pltpu.VMEM_SHARED((NP,), jnp.float32),
            pltpu.VMEM((1, cd), jnp.int32),
            pltpu.VMEM((cd,), jnp.float32),
            pltpu.VMEM((RT,), jnp.float32),
        ],
    )
    def degk(col_hbm, deg_hbm, deg_sh, colidx, ones_buf, stage):
        c = lax.axis_index("c")
        s = lax.axis_index("s")
        node0 = s * RT
        e_base = (c * NS + s) * ept2

        @pl.loop(0, cd // L)
        def _(j):
            ones_buf[pl.ds(j * L, L)] = jnp.ones((L,), jnp.float32)

        @pl.loop(0, RT // L)
        def _(j):
            stage[pl.ds(j * L, L)] = jnp.zeros((L,), jnp.float32)

        pltpu.sync_copy(stage, deg_sh.at[pl.ds(node0, RT)])
        plsc.subcore_barrier()

        @pl.loop(0, nch2)
        def _(ch):
            e0 = e_base + ch * cd
            pltpu.sync_copy(col_hbm.at[pl.ds(e0, cd)], colidx.at[0])
            pltpu.sync_copy(ones_buf, deg_sh.at[colidx.at[0]], add=True)
        plsc.subcore_barrier()

        pltpu.sync_copy(deg_sh.at[pl.ds(node0, RT)], stage)
        pltpu.sync_copy(stage, deg_hbm.at[pl.ds(c * NP + node0, RT)])

    return degk(col_e)


def _tc_prep_body(deg2_ref, out_ref):
    d = deg2_ref[0:1, :] + deg2_ref[1:2, :]
    dv = jnp.where(d > 0.0, jax.lax.rsqrt(jnp.maximum(d, 1e-12)), 0.0)
    out_ref[0:1, :] = dv
    out_ref[1:2, :] = d * EPS


def _tc_prep(deg2):
    out = pl.pallas_call(
        _tc_prep_body,
        out_shape=jax.ShapeDtypeStruct((2, NP), jnp.float32),
    )(deg2.reshape(2, NP))
    return out.reshape(2 * NP)


def _sc_hops(src_pad, row3, col3, dep):
    mesh = plsc.VectorSubcoreMesh(core_axis_name="c", subcore_axis_name="s",
                                  num_cores=1)

    @functools.partial(
        pl.kernel,
        out_type=[
            jax.ShapeDtypeStruct((K * NP, W), jnp.float32),
            jax.ShapeDtypeStruct((NP, W), jnp.float32),
        ],
        mesh=mesh,
        scratch_types=[
            pltpu.VMEM_SHARED((NP, W), jnp.float32),
            pltpu.VMEM((GB, C), jnp.int32),
            pltpu.VMEM((GB, C), jnp.int32),
            pltpu.VMEM((GB, C), jnp.int32),
            pltpu.VMEM((GB, C), jnp.int32),
            pltpu.VMEM((C, W), jnp.float32),
            pltpu.VMEM((C, W), jnp.float32),
            pltpu.VMEM((FB, W), jnp.float32),
            pltpu.VMEM((FB, W), jnp.float32),
            pltpu.VMEM((FB, W), jnp.float32),
            pltpu.VMEM((RT,), jnp.float32),
            pltpu.VMEM((RT,), jnp.float32),
            pltpu.SemaphoreType.DMA,
            pltpu.SemaphoreType.DMA,
            pltpu.SemaphoreType.DMA,
            pltpu.SemaphoreType.DMA,
            pltpu.SemaphoreType.DMA,
        ],
        compiler_params=pltpu.CompilerParams(use_tc_tiling_on_sc=False),
    )
    def hops(src_hbm, row_hbm, col_hbm, dep_hbm, h_hbm, g_hbm, acc_sh,
             rowb0, rowb1, colb0, colb1, rows_buf0, rows_buf1,
             flush_buf, g_buf, zero_buf, dinv_t, epsdeg_t,
             sem_g0, sem_g1, sem_s0, sem_s1, sem_i):
        s = lax.axis_index("s")
        node0 = s * RT
        rbufs = (rows_buf0, rows_buf1)
        gsems = (sem_g0, sem_g1)
        ssems = (sem_s0, sem_s1)
        ibufs = ((rowb0, colb0), (rowb1, colb1))

        @pl.loop(0, FB)
        def _(i):
            for j in range(W // L):
                zero_buf[i, pl.ds(j * L, L)] = jnp.zeros((L,), jnp.float32)

        @pl.loop(0, RT // FB)
        def _(sub):
            pltpu.sync_copy(zero_buf, acc_sh.at[pl.ds(node0 + sub * FB, FB)])

        pltpu.sync_copy(dep_hbm.at[pl.ds(node0, RT)], dinv_t)
        pltpu.sync_copy(dep_hbm.at[pl.ds(NP + node0, RT)], epsdeg_t)

        @pl.loop(0, RT // FB)
        def _(sub):
            r0 = node0 + sub * FB
            pltpu.sync_copy(src_hbm.at[pl.ds(r0, FB)], flush_buf)

            @pl.loop(0, FB // L)
            def _(g):
                dv16 = dinv_t[pl.ds(sub * FB + g * L, L)]
                for i2 in range(L):
                    i = g * L + i2
                    dv = dv16[i2]
                    for j in range(W // L):
                        v = flush_buf[i, pl.ds(j * L, L)]
                        g_buf[i, pl.ds(j * L, L)] = jnp.maximum(v, 0.0) * dv

            pltpu.sync_copy(g_buf, g_hbm.at[pl.ds(r0, FB)])

        def gather_wait(b):
            pltpu.make_async_copy(g_hbm.at[rowb0.at[0]], rbufs[b],
                                  gsems[b]).wait()

        def scatter_wait(b):
            pltpu.make_async_copy(rbufs[b], acc_sh.at[colb0.at[0]],
                                  ssems[b]).wait()

        def idx_wait(half):
            pltpu.make_async_copy(row_hbm.at[s, pl.ds(0, GB)],
                                  ibufs[half][0], sem_i).wait()
            pltpu.make_async_copy(col_hbm.at[s, pl.ds(0, GB)],
                                  ibufs[half][1], sem_i).wait()

        @pl.loop(1, K + 1)
        def _(k):
            plsc.subcore_barrier()

            pltpu.sync_copy(row_hbm.at[s, pl.ds(0, GB)], rowb0)
            pltpu.sync_copy(col_hbm.at[s, pl.ds(0, GB)], colb0)
            pltpu.async_copy(g_hbm.at[rowb0.at[0]], rows_buf0, sem_g0)

            @pl.loop(0, NBLK, step=2)
            def _(blk0):
                for half in range(2):
                    blk = blk0 + half
                    rcur, ccur = ibufs[half]
                    rnxt, cnxt = ibufs[1 - half]
                    for g in range(GB):
                        b = g % 2
                        gather_wait(b)
                        if g == 0:
                            @pl.when(blk > 0)
                            def _():
                                scatter_wait(1 - b)
                        else:
                            scatter_wait(1 - b)
                        if g == 1:
                            @pl.when(blk + 1 < NBLK)
                            def _():
                                pltpu.async_copy(
                                    row_hbm.at[s, pl.ds((blk + 1) * GB, GB)],
                                    rnxt, sem_i)
                                pltpu.async_copy(
                                    col_hbm.at[s, pl.ds((blk + 1) * GB, GB)],
                                    cnxt, sem_i)
                        if g < GB - 1:
                            pltpu.async_copy(g_hbm.at[rcur.at[g + 1]],
                                             rbufs[1 - b], gsems[1 - b])
                        else:
                            @pl.when(blk + 1 < NBLK)
                            def _():
                                idx_wait(1 - half)
                                pltpu.async_copy(g_hbm.at[rnxt.at[0]],
                                                 rbufs[1 - b], gsems[1 - b])
                        pltpu.async_copy(rbufs[b], acc_sh.at[ccur.at[g]],
                                         ssems[b], add=True)

            scatter_wait(1)
            plsc.subcore_barrier()

            @pl.loop(0, RT // FB)
            def _(sub):
                r0 = node0 + sub * FB
                pltpu.sync_copy(acc_sh.at[pl.ds(r0, FB)], flush_buf)
                pltpu.sync_copy(zero_buf, acc_sh.at[pl.ds(r0, FB)])

                @pl.loop(0, FB // L)
                def _(g):
                    dv16 = dinv_t[pl.ds(sub * FB + g * L, L)]
                    ed16 = epsdeg_t[pl.ds(sub * FB + g * L, L)]
                    for i2 in range(L):
                        i = g * L + i2
                        dv = dv16[i2]
                        ed = ed16[i2]
                        for j in range(W // L):
                            h16 = flush_buf[i, pl.ds(j * L, L)] * dv + ed
                            flush_buf[i, pl.ds(j * L, L)] = h16
                            g_buf[i, pl.ds(j * L, L)] = jnp.maximum(h16, 0.0) * dv

                hoff = (k - 1) * NP + r0
                pltpu.sync_copy(flush_buf, h_hbm.at[pl.ds(hoff, FB)])
                pltpu.sync_copy(g_buf, g_hbm.at[pl.ds(r0, FB)])

    return hops(src_pad, row3, col3, dep)


def _pad_rows(h):
    return jnp.zeros((NP, W), h.dtype).at[:N].set(h)


def _combine_body(x_ref, hs_ref, w0_ref, ws_ref,
                  m1w_ref, m1b_ref, m2w_ref, m2b_ref, out_ref, *, relu_out):
    acc = jnp.dot(x_ref[...], w0_ref[...], preferred_element_type=jnp.float32)
    for k in range(K):
        acc += jnp.dot(hs_ref[k], ws_ref[k], preferred_element_type=jnp.float32)
    y = jnp.dot(acc, m1w_ref[...], preferred_element_type=jnp.float32) + m1b_ref[...]
    y = jnp.maximum(y, 0.0)
    o = jnp.dot(y, m2w_ref[...], preferred_element_type=jnp.float32) + m2b_ref[...]
    if relu_out:
        o = jnp.maximum(o, 0.0)
    out_ref[...] = o


def _combine(x, hs, lins, bias, m1w, m1b, bng, bnb, m2w, m2b, relu_out):
    d = lins.shape[1]
    dh = lins.shape[2]
    d2 = m1w.shape[1]
    do = m2w.shape[1]
    rs = jax.lax.rsqrt(jnp.asarray(1.0 + BN_EPS, jnp.float32)) * bng
    m1w_f = m1w * rs[None, :]
    m1b_f = (bias @ m1w) * rs + m1b * rs + bnb
    w0 = lins[0]
    ws = lins[1:]

    grid = (N // BN_ROWS,)
    return pl.pallas_call(
        functools.partial(_combine_body, relu_out=relu_out),
        grid=grid,
        in_specs=[
            pl.BlockSpec((BN_ROWS, d), lambda i: (i, 0)),
            pl.BlockSpec((K, BN_ROWS, d), lambda i: (0, i, 0)),
            pl.BlockSpec((d, dh), lambda i: (0, 0)),
            pl.BlockSpec((K, d, dh), lambda i: (0, 0, 0)),
            pl.BlockSpec((dh, d2), lambda i: (0, 0)),
            pl.BlockSpec((1, d2), lambda i: (0, 0)),
            pl.BlockSpec((d2, do), lambda i: (0, 0)),
            pl.BlockSpec((1, do), lambda i: (0, 0)),
        ],
        out_specs=pl.BlockSpec((BN_ROWS, do), lambda i: (i, 0)),
        out_shape=jax.ShapeDtypeStruct((N, do), jnp.float32),
    )(x, hs, w0, ws, m1w_f, m1b_f.reshape(1, d2), m2w, m2b.reshape(1, do))


def _layer(x, row3, col3, dep, lins, bias, m1w, m1b, bng, bnb, m2w, m2b,
           relu_out):
    h_flat, _ = _sc_hops(_pad_rows(x), row3, col3, dep)
    hs = h_flat.reshape(K, NP, W)[:, :N, :]
    return _combine(x, hs, lins, bias, m1w, m1b, bng, bnb, m2w, m2b, relu_out)


def kernel(x, edge_index, c1_lins, c1_bias, c1_m1w, c1_m1b, c1_bng, c1_bnb,
           c1_m2w, c1_m2b, c2_lins, c2_bias, c2_m1w, c2_m1b, c2_bng, c2_bnb,
           c2_m2w, c2_m2b):
    row_e = edge_index[0]
    col_e = edge_index[1]
    row3 = row_e.reshape(NS, NCH, C)
    col3 = col_e.reshape(NS, NCH, C)
    dep = _tc_prep(_sc_deg(col_e))
    h = _layer(x, row3, col3, dep, c1_lins, c1_bias, c1_m1w, c1_m1b, c1_bng,
               c1_bnb, c1_m2w, c1_m2b, relu_out=True)
    return _layer(h, row3, col3, dep, c2_lins, c2_bias, c2_m1w, c2_m1b,
                  c2_bng, c2_bnb, c2_m2w, c2_m2b, relu_out=False)


# speedup vs baseline: 11.8318x; 1.6847x over previous
import functools
import jax
import jax.numpy as jnp
from jax import lax
from jax.experimental import pallas as pl
from jax.experimental.pallas import tpu as pltpu
from jax.experimental.pallas import tpu_sc as plsc

N = 10000
E = 320000
DIN = 128
DH = 128
DOUT = 64
K = 6
EPS = 1e-07
BN_EPS = 1e-05

NC = 2
NS = 16
L = 16
W = 128

NP = 10240
RT = NP // NS
FB = 32
C = 100
EPT = E // NS
NCH = EPT // C
GB = 10
NBLK = NCH // GB
EPT2 = E // (NC * NS)
NCH2 = EPT2 // C
NBLK2 = NCH2 // GB

BN_ROWS = 1000


def _sc_deg(col_e):
    mesh = plsc.VectorSubcoreMesh(core_axis_name="c", subcore_axis_name="s")
    cd = 80
    ept2 = E // (NC * NS)
    nch2 = ept2 // cd

    @functools.partial(
        pl.kernel,
        out_type=jax.ShapeDtypeStruct((2 * NP,), jnp.float32),
        mesh=mesh,
        scratch_types=[
            pltpu.VMEM_SHARED((NP,), jnp.float32),
            pltpu.VMEM((1, cd), jnp.int32),
            pltpu.VMEM((cd,), jnp.float32),
            pltpu.VMEM((RT,), jnp.float32),
        ],
    )
    def degk(col_hbm, deg_hbm, deg_sh, colidx, ones_buf, stage):
        c = lax.axis_index("c")
        s = lax.axis_index("s")
        node0 = s * RT
        e_base = (c * NS + s) * ept2

        @pl.loop(0, cd // L)
        def _(j):
            ones_buf[pl.ds(j * L, L)] = jnp.ones((L,), jnp.float32)

        @pl.loop(0, RT // L)
        def _(j):
            stage[pl.ds(j * L, L)] = jnp.zeros((L,), jnp.float32)

        pltpu.sync_copy(stage, deg_sh.at[pl.ds(node0, RT)])
        plsc.subcore_barrier()

        @pl.loop(0, nch2)
        def _(ch):
            e0 = e_base + ch * cd
            pltpu.sync_copy(col_hbm.at[pl.ds(e0, cd)], colidx.at[0])
            pltpu.sync_copy(ones_buf, deg_sh.at[colidx.at[0]], add=True)
        plsc.subcore_barrier()

        pltpu.sync_copy(deg_sh.at[pl.ds(node0, RT)], stage)
        pltpu.sync_copy(stage, deg_hbm.at[pl.ds(c * NP + node0, RT)])

    return degk(col_e)


def _tc_prep_body(deg2_ref, out_ref):
    d = deg2_ref[0:1, :] + deg2_ref[1:2, :]
    dv = jnp.where(d > 0.0, jax.lax.rsqrt(jnp.maximum(d, 1e-12)), 0.0)
    out_ref[0:1, :] = dv
    out_ref[1:2, :] = d * EPS


def _tc_prep(deg2):
    out = pl.pallas_call(
        _tc_prep_body,
        out_shape=jax.ShapeDtypeStruct((2, NP), jnp.float32),
    )(deg2.reshape(2, NP))
    return out.reshape(2 * NP)


def _sc_hop(g_pad, row3, col3, zeros_pad):
    mesh = plsc.VectorSubcoreMesh(core_axis_name="c", subcore_axis_name="s")

    @functools.partial(
        pl.kernel,
        out_type=jax.ShapeDtypeStruct((2 * NP, W), jnp.float32),
        mesh=mesh,
        scratch_types=[
            pltpu.VMEM_SHARED((NP, W), jnp.float32),
            pltpu.VMEM((GB, C), jnp.int32),
            pltpu.VMEM((GB, C), jnp.int32),
            pltpu.VMEM((GB, C), jnp.int32),
            pltpu.VMEM((GB, C), jnp.int32),
            pltpu.VMEM((C, W), jnp.float32),
            pltpu.VMEM((C, W), jnp.float32),
            pltpu.SemaphoreType.DMA,
            pltpu.SemaphoreType.DMA,
            pltpu.SemaphoreType.DMA,
            pltpu.SemaphoreType.DMA,
            pltpu.SemaphoreType.DMA,
        ],
        compiler_params=pltpu.CompilerParams(use_tc_tiling_on_sc=False),
    )
    def hop(g_hbm, row_hbm, col_hbm, zeros_hbm, part_hbm, acc_sh,
            rowb0, rowb1, colb0, colb1, rows_buf0, rows_buf1,
            sem_g0, sem_g1, sem_s0, sem_s1, sem_i):
        c = lax.axis_index("c")
        s = lax.axis_index("s")
        w = c * NS + s
        node0 = s * RT
        rbufs = (rows_buf0, rows_buf1)
        gsems = (sem_g0, sem_g1)
        ssems = (sem_s0, sem_s1)
        ibufs = ((rowb0, colb0), (rowb1, colb1))

        def gather_wait(b):
            pltpu.make_async_copy(g_hbm.at[rowb0.at[0]], rbufs[b],
                                  gsems[b]).wait()

        def scatter_wait(b):
            pltpu.make_async_copy(rbufs[b], acc_sh.at[colb0.at[0]],
                                  ssems[b]).wait()

        def idx_wait(half):
            pltpu.make_async_copy(row_hbm.at[w, pl.ds(0, GB)],
                                  ibufs[half][0], sem_i).wait()
            pltpu.make_async_copy(col_hbm.at[w, pl.ds(0, GB)],
                                  ibufs[half][1], sem_i).wait()

        pltpu.sync_copy(zeros_hbm.at[pl.ds(node0, RT)],
                        acc_sh.at[pl.ds(node0, RT)])
        plsc.subcore_barrier()

        pltpu.sync_copy(row_hbm.at[w, pl.ds(0, GB)], rowb0)
        pltpu.sync_copy(col_hbm.at[w, pl.ds(0, GB)], colb0)
        pltpu.async_copy(g_hbm.at[rowb0.at[0]], rows_buf0, sem_g0)

        @pl.loop(0, NBLK2, step=2)
        def _(blk0):
            for half in range(2):
                blk = blk0 + half
                rcur, ccur = ibufs[half]
                rnxt, cnxt = ibufs[1 - half]
                for g in range(GB):
                    b = g % 2
                    gather_wait(b)
                    if g == 0:
                        @pl.when(blk > 0)
                        def _():
                            scatter_wait(1 - b)
                    else:
                        scatter_wait(1 - b)
                    if g == 1:
                        @pl.when(blk + 1 < NBLK2)
                        def _():
                            pltpu.async_copy(
                                row_hbm.at[w, pl.ds((blk + 1) * GB, GB)],
                                rnxt, sem_i)
                            pltpu.async_copy(
                                col_hbm.at[w, pl.ds((blk + 1) * GB, GB)],
                                cnxt, sem_i)
                    if g < GB - 1:
                        pltpu.async_copy(g_hbm.at[rcur.at[g + 1]],
                                         rbufs[1 - b], gsems[1 - b])
                    else:
                        @pl.when(blk + 1 < NBLK2)
                        def _():
                            idx_wait(1 - half)
                            pltpu.async_copy(g_hbm.at[rnxt.at[0]],
                                             rbufs[1 - b], gsems[1 - b])
                    pltpu.async_copy(rbufs[b], acc_sh.at[ccur.at[g]],
                                     ssems[b], add=True)

        scatter_wait(1)
        plsc.subcore_barrier()

        pltpu.sync_copy(acc_sh.at[pl.ds(node0, RT)],
                        part_hbm.at[pl.ds(c * NP + node0, RT)])

    return hop(g_pad, row3, col3, zeros_pad)


def _tc_g0_body(x_ref, dinvb_ref, g_ref):
    g_ref[...] = jnp.maximum(x_ref[...], 0.0) * dinvb_ref[...]


def _tc_g0(x_pad, dinvb):
    bn = 1024
    return pl.pallas_call(
        _tc_g0_body,
        grid=(NP // bn,),
        in_specs=[
            pl.BlockSpec((bn, W), lambda i: (i, 0)),
            pl.BlockSpec((bn, W), lambda i: (i, 0)),
        ],
        out_specs=pl.BlockSpec((bn, W), lambda i: (i, 0)),
        out_shape=jax.ShapeDtypeStruct((NP, W), jnp.float32),
    )(x_pad, dinvb)


def _tc_hop_combine_body(part_ref, dinvb_ref, epsb_ref, h_ref, g_ref):
    p = part_ref[0] + part_ref[1]
    h = p * dinvb_ref[...] + epsb_ref[...]
    h_ref[...] = h
    g_ref[...] = jnp.maximum(h, 0.0) * dinvb_ref[...]


def _tc_hop_combine(part, dinvb, epsb):
    bn = 1024
    return pl.pallas_call(
        _tc_hop_combine_body,
        grid=(NP // bn,),
        in_specs=[
            pl.BlockSpec((2, bn, W), lambda i: (0, i, 0)),
            pl.BlockSpec((bn, W), lambda i: (i, 0)),
            pl.BlockSpec((bn, W), lambda i: (i, 0)),
        ],
        out_specs=[
            pl.BlockSpec((bn, W), lambda i: (i, 0)),
            pl.BlockSpec((bn, W), lambda i: (i, 0)),
        ],
        out_shape=[
            jax.ShapeDtypeStruct((NP, W), jnp.float32),
            jax.ShapeDtypeStruct((NP, W), jnp.float32),
        ],
    )(part.reshape(2, NP, W), dinvb, epsb)


def _pad_rows(h):
    return jnp.zeros((NP, W), h.dtype).at[:N].set(h)


def _combine_body(x_ref, hs_ref, w0_ref, ws_ref,
                  m1w_ref, m1b_ref, m2w_ref, m2b_ref, out_ref, *, relu_out):
    acc = jnp.dot(x_ref[...], w0_ref[...], preferred_element_type=jnp.float32)
    for k in range(K):
        acc += jnp.dot(hs_ref[k], ws_ref[k], preferred_element_type=jnp.float32)
    y = jnp.dot(acc, m1w_ref[...], preferred_element_type=jnp.float32) + m1b_ref[...]
    y = jnp.maximum(y, 0.0)
    o = jnp.dot(y, m2w_ref[...], preferred_element_type=jnp.float32) + m2b_ref[...]
    if relu_out:
        o = jnp.maximum(o, 0.0)
    out_ref[...] = o


def _combine(x, hs, lins, bias, m1w, m1b, bng, bnb, m2w, m2b, relu_out):
    d = lins.shape[1]
    dh = lins.shape[2]
    d2 = m1w.shape[1]
    do = m2w.shape[1]
    rs = jax.lax.rsqrt(jnp.asarray(1.0 + BN_EPS, jnp.float32)) * bng
    m1w_f = m1w * rs[None, :]
    m1b_f = (bias @ m1w) * rs + m1b * rs + bnb
    w0 = lins[0]
    ws = lins[1:]

    grid = (N // BN_ROWS,)
    return pl.pallas_call(
        functools.partial(_combine_body, relu_out=relu_out),
        grid=grid,
        in_specs=[
            pl.BlockSpec((BN_ROWS, d), lambda i: (i, 0)),
            pl.BlockSpec((K, BN_ROWS, d), lambda i: (0, i, 0)),
            pl.BlockSpec((d, dh), lambda i: (0, 0)),
            pl.BlockSpec((K, d, dh), lambda i: (0, 0, 0)),
            pl.BlockSpec((dh, d2), lambda i: (0, 0)),
            pl.BlockSpec((1, d2), lambda i: (0, 0)),
            pl.BlockSpec((d2, do), lambda i: (0, 0)),
            pl.BlockSpec((1, do), lambda i: (0, 0)),
        ],
        out_specs=pl.BlockSpec((BN_ROWS, do), lambda i: (i, 0)),
        out_shape=jax.ShapeDtypeStruct((N, do), jnp.float32),
    )(x, hs, w0, ws, m1w_f, m1b_f.reshape(1, d2), m2w, m2b.reshape(1, do))


def _layer(x_pad, x, row3, col3, zeros_pad, dinvb, epsb, lins, bias,
           m1w, m1b, bng, bnb, m2w, m2b, relu_out):
    g = _tc_g0(x_pad, dinvb)
    hs = []
    for _ in range(K):
        part = _sc_hop(g, row3, col3, zeros_pad)
        h_k, g = _tc_hop_combine(part, dinvb, epsb)
        hs.append(h_k[:N])
    return _combine(x, jnp.stack(hs), lins, bias, m1w, m1b, bng, bnb,
                    m2w, m2b, relu_out)


def kernel(x, edge_index, c1_lins, c1_bias, c1_m1w, c1_m1b, c1_bng, c1_bnb,
           c1_m2w, c1_m2b, c2_lins, c2_bias, c2_m1w, c2_m1b, c2_bng, c2_bnb,
           c2_m2w, c2_m2b):
    row_e = edge_index[0]
    col_e = edge_index[1]
    row3 = row_e.reshape(NC * NS, NCH2, C)
    col3 = col_e.reshape(NC * NS, NCH2, C)
    zeros_pad = jnp.zeros((NP, W), jnp.float32)
    dep = _tc_prep(_sc_deg(col_e))
    dinvb = jnp.broadcast_to(dep[:NP, None], (NP, W))
    epsb = jnp.broadcast_to(dep[NP:, None], (NP, W))
    h = _layer(_pad_rows(x), x, row3, col3, zeros_pad, dinvb, epsb,
               c1_lins, c1_bias, c1_m1w, c1_m1b, c1_bng, c1_bnb, c1_m2w,
               c1_m2b, relu_out=True)
    return _layer(_pad_rows(h), h, row3, col3, zeros_pad, dinvb, epsb,
                  c2_lins, c2_bias, c2_m1w, c2_m1b, c2_bng, c2_bnb, c2_m2w,
                  c2_m2b, relu_out=False)


# speedup vs baseline: 12.7608x; 1.0785x over previous
import functools
import jax
import jax.numpy as jnp
from jax import lax
from jax.experimental import pallas as pl
from jax.experimental.pallas import tpu as pltpu
from jax.experimental.pallas import tpu_sc as plsc

N = 10000
E = 320000
DIN = 128
DH = 128
DOUT = 64
K = 6
EPS = 1e-07
BN_EPS = 1e-05

NC = 2
NS = 16
L = 16
W = 128

NP = 10240
RT = NP // NS
FB = 32
C = 125
EPT = E // NS
NCH = EPT // C
GB = 8
NBLK = NCH // GB
EPT2 = E // (NC * NS)
NCH2 = EPT2 // C
NBLK2 = NCH2 // GB

BN_ROWS = 1000


def _sc_deg(col_e):
    mesh = plsc.VectorSubcoreMesh(core_axis_name="c", subcore_axis_name="s")
    cd = 80
    ept2 = E // (NC * NS)
    nch2 = ept2 // cd

    @functools.partial(
        pl.kernel,
        out_type=jax.ShapeDtypeStruct((2 * NP,), jnp.float32),
        mesh=mesh,
        scratch_types=[
            pltpu.VMEM_SHARED((NP,), jnp.float32),
            pltpu.VMEM((1, cd), jnp.int32),
            pltpu.VMEM((cd,), jnp.float32),
            pltpu.VMEM((RT,), jnp.float32),
        ],
    )
    def degk(col_hbm, deg_hbm, deg_sh, colidx, ones_buf, stage):
        c = lax.axis_index("c")
        s = lax.axis_index("s")
        node0 = s * RT
        e_base = (c * NS + s) * ept2

        @pl.loop(0, cd // L)
        def _(j):
            ones_buf[pl.ds(j * L, L)] = jnp.ones((L,), jnp.float32)

        @pl.loop(0, RT // L)
        def _(j):
            stage[pl.ds(j * L, L)] = jnp.zeros((L,), jnp.float32)

        pltpu.sync_copy(stage, deg_sh.at[pl.ds(node0, RT)])
        plsc.subcore_barrier()

        @pl.loop(0, nch2)
        def _(ch):
            e0 = e_base + ch * cd
            pltpu.sync_copy(col_hbm.at[pl.ds(e0, cd)], colidx.at[0])
            pltpu.sync_copy(ones_buf, deg_sh.at[colidx.at[0]], add=True)
        plsc.subcore_barrier()

        pltpu.sync_copy(deg_sh.at[pl.ds(node0, RT)], stage)
        pltpu.sync_copy(stage, deg_hbm.at[pl.ds(c * NP + node0, RT)])

    return degk(col_e)


def _tc_prep_body(deg2_ref, out_ref):
    d = deg2_ref[0:1, :] + deg2_ref[1:2, :]
    dv = jnp.where(d > 0.0, jax.lax.rsqrt(jnp.maximum(d, 1e-12)), 0.0)
    out_ref[0:1, :] = dv
    out_ref[1:2, :] = d * EPS


def _tc_prep(deg2):
    out = pl.pallas_call(
        _tc_prep_body,
        out_shape=jax.ShapeDtypeStruct((2, NP), jnp.float32),
    )(deg2.reshape(2, NP))
    return out.reshape(2 * NP)


def _sc_hop(g_pad, row3, col3, zeros_pad):
    mesh = plsc.VectorSubcoreMesh(core_axis_name="c", subcore_axis_name="s")

    @functools.partial(
        pl.kernel,
        out_type=jax.ShapeDtypeStruct((2 * NP, W), jnp.float32),
        mesh=mesh,
        scratch_types=[
            pltpu.VMEM_SHARED((NP, W), jnp.float32),
            pltpu.VMEM((GB, C), jnp.int32),
            pltpu.VMEM((GB, C), jnp.int32),
            pltpu.VMEM((GB, C), jnp.int32),
            pltpu.VMEM((GB, C), jnp.int32),
            pltpu.VMEM((C, W), jnp.float32),
            pltpu.VMEM((C, W), jnp.float32),
            pltpu.SemaphoreType.DMA,
            pltpu.SemaphoreType.DMA,
            pltpu.SemaphoreType.DMA,
            pltpu.SemaphoreType.DMA,
            pltpu.SemaphoreType.DMA,
        ],
        compiler_params=pltpu.CompilerParams(use_tc_tiling_on_sc=False),
    )
    def hop(g_hbm, row_hbm, col_hbm, zeros_hbm, part_hbm, acc_sh,
            rowb0, rowb1, colb0, colb1, rows_buf0, rows_buf1,
            sem_g0, sem_g1, sem_s0, sem_s1, sem_i):
        c = lax.axis_index("c")
        s = lax.axis_index("s")
        w = c * NS + s
        node0 = s * RT
        rbufs = (rows_buf0, rows_buf1)
        gsems = (sem_g0, sem_g1)
        ssems = (sem_s0, sem_s1)
        ibufs = ((rowb0, colb0), (rowb1, colb1))

        def gather_wait(b):
            pltpu.make_async_copy(g_hbm.at[rowb0.at[0]], rbufs[b],
                                  gsems[b]).wait()

        def scatter_wait(b):
            pltpu.make_async_copy(rbufs[b], acc_sh.at[colb0.at[0]],
                                  ssems[b]).wait()

        def idx_wait(half):
            pltpu.make_async_copy(row_hbm.at[w, pl.ds(0, GB)],
                                  ibufs[half][0], sem_i).wait()
            pltpu.make_async_copy(col_hbm.at[w, pl.ds(0, GB)],
                                  ibufs[half][1], sem_i).wait()

        zdesc = pltpu.async_copy(zeros_hbm.at[pl.ds(node0, RT)],
                                 acc_sh.at[pl.ds(node0, RT)], sem_i)
        pltpu.sync_copy(row_hbm.at[w, pl.ds(0, GB)], rowb0)
        pltpu.sync_copy(col_hbm.at[w, pl.ds(0, GB)], colb0)
        zdesc.wait()
        plsc.subcore_barrier()
        pltpu.async_copy(g_hbm.at[rowb0.at[0]], rows_buf0, sem_g0)

        @pl.loop(0, NBLK2, step=2)
        def _(blk0):
            for half in range(2):
                blk = blk0 + half
                rcur, ccur = ibufs[half]
                rnxt, cnxt = ibufs[1 - half]
                for g in range(GB):
                    b = g % 2
                    gather_wait(b)
                    if g == 0:
                        @pl.when(blk > 0)
                        def _():
                            scatter_wait(1 - b)
                    else:
                        scatter_wait(1 - b)
                    if g == 1:
                        @pl.when(blk + 1 < NBLK2)
                        def _():
                            pltpu.async_copy(
                                row_hbm.at[w, pl.ds((blk + 1) * GB, GB)],
                                rnxt, sem_i)
                            pltpu.async_copy(
                                col_hbm.at[w, pl.ds((blk + 1) * GB, GB)],
                                cnxt, sem_i)
                    if g < GB - 1:
                        pltpu.async_copy(g_hbm.at[rcur.at[g + 1]],
                                         rbufs[1 - b], gsems[1 - b])
                    else:
                        @pl.when(blk + 1 < NBLK2)
                        def _():
                            idx_wait(1 - half)
                            pltpu.async_copy(g_hbm.at[rnxt.at[0]],
                                             rbufs[1 - b], gsems[1 - b])
                    pltpu.async_copy(rbufs[b], acc_sh.at[ccur.at[g]],
                                     ssems[b], add=True)

        scatter_wait(1)
        plsc.subcore_barrier()

        pltpu.sync_copy(acc_sh.at[pl.ds(node0, RT)],
                        part_hbm.at[pl.ds(c * NP + node0, RT)])

    return hop(g_pad, row3, col3, zeros_pad)


def _tc_g0_body(x_ref, dinvb_ref, g_ref):
    g_ref[...] = jnp.maximum(x_ref[...], 0.0) * dinvb_ref[...]


def _tc_g0(x_pad, dinvb):
    bn = 1024
    return pl.pallas_call(
        _tc_g0_body,
        grid=(NP // bn,),
        in_specs=[
            pl.BlockSpec((bn, W), lambda i: (i, 0)),
            pl.BlockSpec((bn, W), lambda i: (i, 0)),
        ],
        out_specs=pl.BlockSpec((bn, W), lambda i: (i, 0)),
        out_shape=jax.ShapeDtypeStruct((NP, W), jnp.float32),
    )(x_pad, dinvb)


def _tc_hop_combine_body(part_ref, dinvb_ref, epsb_ref, h_ref, g_ref):
    p = part_ref[0] + part_ref[1]
    h = p * dinvb_ref[...] + epsb_ref[...]
    h_ref[...] = h
    g_ref[...] = jnp.maximum(h, 0.0) * dinvb_ref[...]


def _tc_hop_combine(part, dinvb, epsb):
    bn = 1024
    return pl.pallas_call(
        _tc_hop_combine_body,
        grid=(NP // bn,),
        in_specs=[
            pl.BlockSpec((2, bn, W), lambda i: (0, i, 0)),
            pl.BlockSpec((bn, W), lambda i: (i, 0)),
            pl.BlockSpec((bn, W), lambda i: (i, 0)),
        ],
        out_specs=[
            pl.BlockSpec((bn, W), lambda i: (i, 0)),
            pl.BlockSpec((bn, W), lambda i: (i, 0)),
        ],
        out_shape=[
            jax.ShapeDtypeStruct((NP, W), jnp.float32),
            jax.ShapeDtypeStruct((NP, W), jnp.float32),
        ],
    )(part.reshape(2, NP, W), dinvb, epsb)


def _pad_rows(h):
    return jnp.zeros((NP, W), h.dtype).at[:N].set(h)


def _combine_body(x_ref, hs_ref, w0_ref, ws_ref,
                  m1w_ref, m1b_ref, m2w_ref, m2b_ref, out_ref, *, relu_out):
    acc = jnp.dot(x_ref[...], w0_ref[...], preferred_element_type=jnp.float32)
    for k in range(K):
        acc += jnp.dot(hs_ref[k], ws_ref[k], preferred_element_type=jnp.float32)
    y = jnp.dot(acc, m1w_ref[...], preferred_element_type=jnp.float32) + m1b_ref[...]
    y = jnp.maximum(y, 0.0)
    o = jnp.dot(y, m2w_ref[...], preferred_element_type=jnp.float32) + m2b_ref[...]
    if relu_out:
        o = jnp.maximum(o, 0.0)
    out_ref[...] = o


def _combine(x, hs, lins, bias, m1w, m1b, bng, bnb, m2w, m2b, relu_out):
    d = lins.shape[1]
    dh = lins.shape[2]
    d2 = m1w.shape[1]
    do = m2w.shape[1]
    rs = jax.lax.rsqrt(jnp.asarray(1.0 + BN_EPS, jnp.float32)) * bng
    m1w_f = m1w * rs[None, :]
    m1b_f = (bias @ m1w) * rs + m1b * rs + bnb
    w0 = lins[0]
    ws = lins[1:]

    grid = (N // BN_ROWS,)
    return pl.pallas_call(
        functools.partial(_combine_body, relu_out=relu_out),
        grid=grid,
        in_specs=[
            pl.BlockSpec((BN_ROWS, d), lambda i: (i, 0)),
            pl.BlockSpec((K, BN_ROWS, d), lambda i: (0, i, 0)),
            pl.BlockSpec((d, dh), lambda i: (0, 0)),
            pl.BlockSpec((K, d, dh), lambda i: (0, 0, 0)),
            pl.BlockSpec((dh, d2), lambda i: (0, 0)),
            pl.BlockSpec((1, d2), lambda i: (0, 0)),
            pl.BlockSpec((d2, do), lambda i: (0, 0)),
            pl.BlockSpec((1, do), lambda i: (0, 0)),
        ],
        out_specs=pl.BlockSpec((BN_ROWS, do), lambda i: (i, 0)),
        out_shape=jax.ShapeDtypeStruct((N, do), jnp.float32),
    )(x, hs, w0, ws, m1w_f, m1b_f.reshape(1, d2), m2w, m2b.reshape(1, do))


def _layer(x_pad, x, row3, col3, zeros_pad, dinvb, epsb, lins, bias,
           m1w, m1b, bng, bnb, m2w, m2b, relu_out):
    g = _tc_g0(x_pad, dinvb)
    hs = []
    for _ in range(K):
        part = _sc_hop(g, row3, col3, zeros_pad)
        h_k, g = _tc_hop_combine(part, dinvb, epsb)
        hs.append(h_k[:N])
    return _combine(x, jnp.stack(hs), lins, bias, m1w, m1b, bng, bnb,
                    m2w, m2b, relu_out)


def kernel(x, edge_index, c1_lins, c1_bias, c1_m1w, c1_m1b, c1_bng, c1_bnb,
           c1_m2w, c1_m2b, c2_lins, c2_bias, c2_m1w, c2_m1b, c2_bng, c2_bnb,
           c2_m2w, c2_m2b):
    row_e = edge_index[0]
    col_e = edge_index[1]
    row3 = row_e.reshape(NC * NS, NCH2, C)
    col3 = col_e.reshape(NC * NS, NCH2, C)
    zeros_pad = jnp.zeros((NP, W), jnp.float32)
    dep = _tc_prep(_sc_deg(col_e))
    dinvb = jnp.broadcast_to(dep[:NP, None], (NP, W))
    epsb = jnp.broadcast_to(dep[NP:, None], (NP, W))
    h = _layer(_pad_rows(x), x, row3, col3, zeros_pad, dinvb, epsb,
               c1_lins, c1_bias, c1_m1w, c1_m1b, c1_bng, c1_bnb, c1_m2w,
               c1_m2b, relu_out=True)
    return _layer(_pad_rows(h), h, row3, col3, zeros_pad, dinvb, epsb,
                  c2_lins, c2_bias, c2_m1w, c2_m1b, c2_bng, c2_bnb, c2_m2w,
                  c2_m2b, relu_out=False)
